# Initial kernel scaffold; baseline (speedup 1.0000x reference)
import jax
import jax.numpy as jnp
from jax.experimental import pallas as pl


def kernel(x_main, x_mod, csr_idx):
    raise NotImplementedError("write your pallas kernel here")


# speedup vs baseline: 42.1188x; 42.1188x over previous
import functools

import jax
import jax.numpy as jnp
from jax import lax
from jax.experimental import pallas as pl
from jax.experimental.pallas import tpu as pltpu
from jax.experimental.pallas import tpu_sc as plsc

N_NODES = 10000
N_EDGES = 320000
D = 128
NLANE = 16
NQ = D // NLANE

NC = 2
NS = 16
NW = NC * NS
S = 320
CSR_SLICE = S + 32
CSR_PAD_LEN = (NW - 1) * S + CSR_SLICE
C = 128
SEARCH_STEPS = 9


def _extract(vec_ref, idx):
    return vec_ref[pl.ds(idx, NLANE)][0]


_mesh = plsc.VectorSubcoreMesh(core_axis_name="c", subcore_axis_name="s")


@functools.partial(
    pl.kernel,
    out_type=jax.ShapeDtypeStruct((NW * S, D), jnp.float32),
    mesh=_mesh,
    scratch_types=[
        pltpu.VMEM((CSR_SLICE,), jnp.int32),
        pltpu.VMEM((C, D), jnp.float32),
        pltpu.VMEM((S + 1, D), jnp.float32),
        pltpu.VMEM((C + NLANE,), jnp.int32),
    ],
    compiler_params=pltpu.CompilerParams(needs_layout_passes=False),
)
def _csr_pool(csr_hbm, xmod_hbm, out_hbm, csr_v, buf_v, outb_v, seg_v):
    wid = lax.axis_index("s") * NC + lax.axis_index("c")
    j0 = wid * S
    pltpu.sync_copy(csr_hbm.at[pl.ds(j0, CSR_SLICE)], csr_v)

    zero = jnp.zeros((NLANE,), jnp.float32)
    lane = lax.iota(jnp.int32, NLANE)

    def zinit(j, carry):
        for q in range(NQ):
            outb_v[j, pl.ds(q * NLANE, NLANE)] = zero
        return carry

    lax.fori_loop(0, S + 1, zinit, 0)

    e0 = _extract(csr_v, 0)
    e_end = _extract(csr_v, S)
    k0 = e0 // C
    k1 = (e_end + (C - 1)) // C
    e0v = jnp.full((NLANE,), e0, jnp.int32)
    e_endv = jnp.full((NLANE,), e_end, jnp.int32)

    def chunk_body(k, carry):
        pltpu.sync_copy(xmod_hbm.at[pl.ds(k * C, C)], buf_v)
        base = k * C

        def search_body(g, carry_):
            i16 = jnp.full((NLANE,), base + g * NLANE, jnp.int32) + lane
            lo = jnp.zeros((NLANE,), jnp.int32)
            hi = jnp.full((NLANE,), S, jnp.int32)
            for _ in range(SEARCH_STEPS):
                mid = (lo + hi + 1) >> 1
                probe = plsc.load_gather(csr_v, [mid])
                le = probe <= i16
                lo = jnp.where(le, mid, lo)
                hi = jnp.where(le, hi, mid - 1)
            valid = (i16 >= e0v) & (i16 < e_endv)
            js16 = jnp.where(valid, lo, jnp.full((NLANE,), S, jnp.int32))
            seg_v[pl.ds(g * NLANE, NLANE)] = js16
            return carry_

        lax.fori_loop(0, C // NLANE, search_body, 0)

        def row_body(r, carry_):
            js = _extract(seg_v, r)
            rowv = jnp.full((NLANE,), js, jnp.int32)
            for q in range(NQ):
                col = lane + (q * NLANE)
                v = buf_v[r, pl.ds(q * NLANE, NLANE)]
                plsc.addupdate_scatter(outb_v, [rowv, col], v)
            return carry_

        lax.fori_loop(0, C, row_body, 0)
        return carry

    lax.fori_loop(k0, k1, chunk_body, 0)

    pltpu.sync_copy(outb_v.at[pl.ds(0, S)], out_hbm.at[pl.ds(j0, S)])


def kernel(x_main, x_mod, csr_idx):
    del x_main
    csr = csr_idx.astype(jnp.int32)
    pad = jnp.full((CSR_PAD_LEN - (N_NODES + 1),), N_EDGES, jnp.int32)
    csr_pad = jnp.concatenate([csr, pad])
    out_pad = _csr_pool(csr_pad, x_mod)
    return out_pad[:N_NODES]
